# baseline (speedup 1.0000x reference)
import functools

import jax
import jax.numpy as jnp
from jax import lax
from jax.experimental import pallas as pl
from jax.experimental.pallas import tpu as pltpu
from jax.experimental.pallas import tpu_sc as plsc

_NUM_WORKERS = 32
_CHUNK = 8


def _sc_prompt_fill(table, labels, pre, tail, row_elems, gen_elems):
    b = labels.shape[0]
    d = pre.shape[0]
    tail_elems = tail.shape[0]
    b_per_w = b // _NUM_WORKERS
    n_chunks = b_per_w // _CHUNK

    mesh = plsc.VectorSubcoreMesh(core_axis_name="c", subcore_axis_name="s")

    @functools.partial(
        pl.kernel,
        mesh=mesh,
        out_type=jax.ShapeDtypeStruct((b * row_elems,), jnp.float32),
        scratch_types=[
            pltpu.VMEM((b_per_w,), jnp.int32),
            pltpu.VMEM((d,), jnp.float32),
            pltpu.VMEM((tail_elems,), jnp.float32),
            pltpu.VMEM((_CHUNK, gen_elems), jnp.float32),
            pltpu.SemaphoreType.DMA,
            pltpu.SemaphoreType.DMA,
        ],
    )
    def k(table_hbm, label_hbm, pre_hbm, tail_hbm, out_hbm,
          idx_v, pre_v, tail_v, rows_v, gsem, wsem):
        wid = lax.axis_index("s") * 2 + lax.axis_index("c")
        base = wid * b_per_w
        pltpu.sync_copy(label_hbm.at[pl.ds(base, b_per_w)], idx_v)
        pltpu.sync_copy(pre_hbm, pre_v)
        pltpu.sync_copy(tail_hbm, tail_v)
        for c in range(n_chunks):
            pltpu.async_copy(
                table_hbm.at[idx_v.at[pl.ds(c * _CHUNK, _CHUNK)]],
                rows_v, gsem).wait()
            waits = []
            for r in range(_CHUNK):
                off = (base + c * _CHUNK + r) * row_elems
                d1 = pltpu.make_async_copy(
                    pre_v, out_hbm.at[pl.ds(off, d)], wsem)
                d2 = pltpu.make_async_copy(
                    rows_v.at[r], out_hbm.at[pl.ds(off + d, gen_elems)], wsem)
                d3 = pltpu.make_async_copy(
                    tail_v, out_hbm.at[pl.ds(off + d + gen_elems, tail_elems)],
                    wsem)
                d1.start()
                d2.start()
                d3.start()
                waits += [d1, d2, d3]
            for dsc in waits:
                dsc.wait()

    return k(table, labels, pre, tail)


def kernel(label, ctx_generic, ctx_modality, ctx_platform,
           token_prefix, token_suffix):
    b = label.shape[0]
    num_class, n_gen, d = ctx_generic.shape
    n_zero = ctx_modality.shape[1] + ctx_platform.shape[1]
    n_suf = token_suffix.shape[1]
    seq = token_prefix.shape[1] + n_gen + n_zero + n_suf

    table = ctx_generic.reshape(num_class, n_gen * d)
    pre = token_prefix.reshape(-1).astype(jnp.float32)
    tail = jnp.concatenate([
        jnp.zeros((n_zero * d,), jnp.float32),
        token_suffix.reshape(-1).astype(jnp.float32),
    ])
    flat = _sc_prompt_fill(table, label.astype(jnp.int32), pre, tail,
                           seq * d, n_gen * d)
    return flat.reshape(b, seq, d)


# speedup vs baseline: 7.4465x; 7.4465x over previous
import functools

import jax
import jax.numpy as jnp
from jax import lax
from jax.experimental import pallas as pl
from jax.experimental.pallas import tpu as pltpu
from jax.experimental.pallas import tpu_sc as plsc

_NUM_WORKERS = 32
_CHUNK = 8
_LANES = 16


def _sc_prompt_fill(table, labels, template):
    b = labels.shape[0]
    _, n_gen, d = table.shape
    seq = template.shape[1]
    b_per_w = b // _NUM_WORKERS
    n_chunks = b_per_w // _CHUNK

    mesh = plsc.VectorSubcoreMesh(core_axis_name="c", subcore_axis_name="s")

    @functools.partial(
        pl.kernel,
        mesh=mesh,
        out_type=jax.ShapeDtypeStruct((b, seq, d), jnp.float32),
        scratch_types=[
            pltpu.VMEM((b_per_w,), jnp.int32),
            pltpu.VMEM((_CHUNK, n_gen, d), jnp.float32),
            pltpu.VMEM((1, seq, d), jnp.float32),
            pltpu.VMEM((1, seq, d), jnp.float32),
            pltpu.SemaphoreType.DMA,
            pltpu.SemaphoreType.DMA,
            pltpu.SemaphoreType.DMA,
        ],
    )
    def k(table_hbm, label_hbm, tmpl_hbm, out_hbm,
          idx_v, stage_v, row0_v, row1_v, gsem, wsem0, wsem1):
        wid = lax.axis_index("s") * 2 + lax.axis_index("c")
        base = pl.multiple_of(wid * b_per_w, b_per_w)
        pltpu.sync_copy(label_hbm.at[pl.ds(base, b_per_w)], idx_v)
        pltpu.sync_copy(tmpl_hbm, row0_v)
        pltpu.sync_copy(tmpl_hbm, row1_v)
        rows = (row0_v, row1_v)
        sems = (wsem0, wsem1)

        def chunk_body(c, carry):
            coff = pl.multiple_of(c * _CHUNK, _CHUNK)
            pltpu.async_copy(
                table_hbm.at[idx_v.at[pl.ds(coff, _CHUNK)]],
                stage_v, gsem).wait()
            started = []
            for r in range(_CHUNK):
                p = r % 2
                if r >= 2:
                    started[r - 2].wait()
                for srow in range(n_gen):
                    for j in range(d // _LANES):
                        rows[p][0, srow + 1, pl.ds(j * _LANES, _LANES)] = (
                            stage_v[r, srow, pl.ds(j * _LANES, _LANES)])
                dsc = pltpu.make_async_copy(
                    rows[p], out_hbm.at[pl.ds(base + coff + r, 1)], sems[p])
                dsc.start()
                started.append(dsc)
            started[_CHUNK - 2].wait()
            started[_CHUNK - 1].wait()
            return carry

        lax.fori_loop(0, n_chunks, chunk_body, 0)

    return k(table, labels, template)


def kernel(label, ctx_generic, ctx_modality, ctx_platform,
           token_prefix, token_suffix):
    n_gen = ctx_generic.shape[1]
    d = ctx_generic.shape[2]
    n_zero = ctx_modality.shape[1] + ctx_platform.shape[1]

    template = jnp.concatenate([
        token_prefix.astype(jnp.float32),
        jnp.zeros((1, n_gen + n_zero, d), jnp.float32),
        token_suffix.astype(jnp.float32),
    ], axis=1)
    return _sc_prompt_fill(ctx_generic, label.astype(jnp.int32), template)


# speedup vs baseline: 13.2375x; 1.7777x over previous
import functools

import jax
import jax.numpy as jnp
from jax import lax
from jax.experimental import pallas as pl
from jax.experimental.pallas import tpu as pltpu
from jax.experimental.pallas import tpu_sc as plsc

_NUM_WORKERS = 32
_NSUB = 16
_CHUNK = 8
_TW = 16
_LANES = 16


def _sc_prompt_fill(table, labels, template):
    b = labels.shape[0]
    _, n_gen, d = table.shape
    n_tmpl = template.shape[0]
    seq = n_tmpl + n_gen
    b_per_w = b // _NUM_WORKERS
    n_chunks = b_per_w // _CHUNK
    lanes_per_row = d // _LANES

    mesh = plsc.VectorSubcoreMesh(core_axis_name="c", subcore_axis_name="s")

    @functools.partial(
        pl.kernel,
        mesh=mesh,
        out_type=jax.ShapeDtypeStruct((seq, b, d), jnp.float32),
        scratch_types=[
            pltpu.VMEM((b_per_w,), jnp.int32),
            pltpu.VMEM((_CHUNK, n_gen, d), jnp.float32),
            pltpu.VMEM((n_gen, _CHUNK, d), jnp.float32),
            pltpu.VMEM_SHARED((n_tmpl, _TW, d), jnp.float32),
            pltpu.SemaphoreType.DMA,
            pltpu.SemaphoreType.DMA,
            pltpu.SemaphoreType.DMA,
        ],
    )
    def k(table_hbm, label_hbm, tmpl_hbm, out_hbm,
          idx_v, stage_v, genbuf_v, shared, gsem, wsem, tsem):
        cid = lax.axis_index("c")
        sid = lax.axis_index("s")
        wid = sid * 2 + cid
        base = pl.multiple_of(wid * b_per_w, b_per_w)
        pltpu.sync_copy(label_hbm.at[pl.ds(base, b_per_w)], idx_v)

        n_big = n_tmpl - (n_tmpl // _NSUB) * _NSUB
        n_small = n_tmpl // _NSUB

        @pl.when(sid < n_big)
        def _load_big():
            s0 = sid * (n_small + 1)
            pltpu.sync_copy(tmpl_hbm.at[pl.ds(s0, n_small + 1)],
                            shared.at[pl.ds(s0, n_small + 1)])

        @pl.when(jnp.logical_not(sid < n_big))
        def _load_small():
            s0 = n_big * (n_small + 1) + (sid - n_big) * n_small
            pltpu.sync_copy(tmpl_hbm.at[pl.ds(s0, n_small)],
                            shared.at[pl.ds(s0, n_small)])

        plsc.subcore_barrier()

        for kk in range(n_tmpl):
            t = 0 if kk == 0 else kk + n_gen
            for h in range(b_per_w // _TW):
                pltpu.make_async_copy(
                    shared.at[pl.ds(kk, 1)],
                    out_hbm.at[pl.ds(t, 1), pl.ds(base + h * _TW, _TW), :],
                    tsem).start()

        def chunk_body(c, carry):
            coff = pl.multiple_of(c * _CHUNK, _CHUNK)
            pltpu.async_copy(
                table_hbm.at[idx_v.at[pl.ds(coff, _CHUNK)]],
                stage_v, gsem).wait()

            @pl.when(c > 0)
            def _drain_prev():
                pltpu.make_async_copy(
                    table_hbm.at[pl.ds(0, _CHUNK)], genbuf_v, wsem).wait()

            for r in range(_CHUNK):
                for j in range(n_gen):
                    for l in range(lanes_per_row):
                        genbuf_v[j, r, pl.ds(l * _LANES, _LANES)] = (
                            stage_v[r, j, pl.ds(l * _LANES, _LANES)])
            for j in range(n_gen):
                pltpu.make_async_copy(
                    genbuf_v.at[pl.ds(j, 1)],
                    out_hbm.at[pl.ds(1 + j, 1),
                               pl.ds(base + coff, _CHUNK), :],
                    wsem).start()
            return carry

        lax.fori_loop(0, n_chunks, chunk_body, 0)

        pltpu.make_async_copy(
            table_hbm.at[pl.ds(0, _CHUNK)], genbuf_v, wsem).wait()
        for _ in range(b_per_w // _TW):
            pltpu.make_async_copy(tmpl_hbm, shared, tsem).wait()

    return k(table, labels, template)


def kernel(label, ctx_generic, ctx_modality, ctx_platform,
           token_prefix, token_suffix):
    b = label.shape[0]
    n_gen = ctx_generic.shape[1]
    d = ctx_generic.shape[2]
    n_zero = ctx_modality.shape[1] + ctx_platform.shape[1]
    template = jnp.concatenate([
        jnp.broadcast_to(token_prefix.astype(jnp.float32),
                         (token_prefix.shape[1], _TW, d)),
        jnp.zeros((n_zero, _TW, d), jnp.float32),
        jnp.broadcast_to(
            jnp.transpose(token_suffix.astype(jnp.float32), (1, 0, 2)),
            (token_suffix.shape[1], _TW, d)),
    ], axis=0)
    slabbed = _sc_prompt_fill(ctx_generic, label.astype(jnp.int32), template)
    return jnp.transpose(slabbed, (1, 0, 2))


# speedup vs baseline: 13.9853x; 1.0565x over previous
import functools

import jax
import jax.numpy as jnp
from jax import lax
from jax.experimental import pallas as pl
from jax.experimental.pallas import tpu as pltpu
from jax.experimental.pallas import tpu_sc as plsc

_NUM_WORKERS = 32
_NSUB = 16
_CHUNK = 8
_TW = 16
_LANES = 16


def _sc_prompt_fill(table, labels, template):
    b = labels.shape[0]
    _, n_gen, d = table.shape
    n_tmpl = template.shape[0]
    seq = n_tmpl + n_gen
    b_per_w = b // _NUM_WORKERS
    n_chunks = b_per_w // _CHUNK
    lanes_per_row = d // _LANES

    mesh = plsc.VectorSubcoreMesh(core_axis_name="c", subcore_axis_name="s")

    @functools.partial(
        pl.kernel,
        mesh=mesh,
        out_type=jax.ShapeDtypeStruct((seq, b, d), jnp.float32),
        scratch_types=[
            pltpu.VMEM((b_per_w,), jnp.int32),
            pltpu.VMEM((_CHUNK, n_gen, d), jnp.float32),
            pltpu.VMEM((n_gen, _CHUNK, d), jnp.float32),
            pltpu.VMEM_SHARED((n_tmpl, _TW, d), jnp.float32),
            pltpu.SemaphoreType.DMA,
            pltpu.SemaphoreType.DMA,
            pltpu.SemaphoreType.DMA,
        ],
    )
    def k(table_hbm, label_hbm, tmpl_hbm, out_hbm,
          idx_v, stage_v, genbuf_v, shared, gsem, wsem, tsem):
        cid = lax.axis_index("c")
        sid = lax.axis_index("s")
        wid = sid * 2 + cid
        base = pl.multiple_of(wid * b_per_w, b_per_w)
        pltpu.sync_copy(label_hbm.at[pl.ds(base, b_per_w)], idx_v)

        n_big = n_tmpl - (n_tmpl // _NSUB) * _NSUB
        n_small = n_tmpl // _NSUB

        @pl.when(sid < n_big)
        def _load_big():
            s0 = sid * (n_small + 1)
            pltpu.sync_copy(tmpl_hbm.at[pl.ds(s0, n_small + 1)],
                            shared.at[pl.ds(s0, n_small + 1)])

        @pl.when(jnp.logical_not(sid < n_big))
        def _load_small():
            s0 = n_big * (n_small + 1) + (sid - n_big) * n_small
            pltpu.sync_copy(tmpl_hbm.at[pl.ds(s0, n_small)],
                            shared.at[pl.ds(s0, n_small)])

        plsc.subcore_barrier()

        for h in range(b_per_w // _TW):
            dst_b = pl.ds(base + h * _TW, _TW)
            pltpu.make_async_copy(
                shared.at[pl.ds(0, 1)],
                out_hbm.at[pl.ds(0, 1), dst_b, :], tsem).start()
            pltpu.make_async_copy(
                shared.at[pl.ds(1, n_tmpl - 1)],
                out_hbm.at[pl.ds(1 + n_gen, n_tmpl - 1), dst_b, :],
                tsem).start()

        def chunk_body(c, carry):
            coff = pl.multiple_of(c * _CHUNK, _CHUNK)
            pltpu.async_copy(
                table_hbm.at[idx_v.at[pl.ds(coff, _CHUNK)]],
                stage_v, gsem).wait()

            @pl.when(c > 0)
            def _drain_prev():
                pltpu.make_async_copy(
                    table_hbm.at[pl.ds(0, _CHUNK)], genbuf_v, wsem).wait()

            for r in range(_CHUNK):
                for j in range(n_gen):
                    for l in range(lanes_per_row):
                        genbuf_v[j, r, pl.ds(l * _LANES, _LANES)] = (
                            stage_v[r, j, pl.ds(l * _LANES, _LANES)])
            pltpu.make_async_copy(
                genbuf_v,
                out_hbm.at[pl.ds(1, n_gen), pl.ds(base + coff, _CHUNK), :],
                wsem).start()
            return carry

        lax.fori_loop(0, n_chunks, chunk_body, 0)

        pltpu.make_async_copy(
            table_hbm.at[pl.ds(0, _CHUNK)], genbuf_v, wsem).wait()
        for _ in range(b_per_w // _TW):
            pltpu.make_async_copy(tmpl_hbm, shared, tsem).wait()

    return k(table, labels, template)


def kernel(label, ctx_generic, ctx_modality, ctx_platform,
           token_prefix, token_suffix):
    b = label.shape[0]
    n_gen = ctx_generic.shape[1]
    d = ctx_generic.shape[2]
    n_zero = ctx_modality.shape[1] + ctx_platform.shape[1]
    template = jnp.concatenate([
        jnp.broadcast_to(token_prefix.astype(jnp.float32),
                         (token_prefix.shape[1], _TW, d)),
        jnp.zeros((n_zero, _TW, d), jnp.float32),
        jnp.broadcast_to(
            jnp.transpose(token_suffix.astype(jnp.float32), (1, 0, 2)),
            (token_suffix.shape[1], _TW, d)),
    ], axis=0)
    slabbed = _sc_prompt_fill(ctx_generic, label.astype(jnp.int32), template)
    return jnp.transpose(slabbed, (1, 0, 2))


# speedup vs baseline: 16.5238x; 1.1815x over previous
import functools

import jax
import jax.numpy as jnp
from jax import lax
from jax.experimental import pallas as pl
from jax.experimental.pallas import tpu as pltpu
from jax.experimental.pallas import tpu_sc as plsc

_NUM_WORKERS = 32
_NSUB = 16
_CHUNK = 8
_TW = 16
_BW = 32
_LANES = 16


def _sc_prompt_fill(table, labels, tmpl5, crows):
    b = labels.shape[0]
    _, n_gen, d = table.shape
    n_sh = tmpl5.shape[0]
    n_own_tot = crows.shape[0]
    seq = 1 + n_gen + (n_sh - 1) + n_own_tot
    b_per_w = b // _NUM_WORKERS
    n_chunks = b_per_w // _CHUNK
    lanes_per_row = d // _LANES
    own_t0 = seq - n_own_tot

    mesh = plsc.VectorSubcoreMesh(core_axis_name="c", subcore_axis_name="s")

    @functools.partial(
        pl.kernel,
        mesh=mesh,
        out_type=jax.ShapeDtypeStruct((seq, b, d), jnp.float32),
        scratch_types=[
            pltpu.VMEM((b_per_w,), jnp.int32),
            pltpu.VMEM((2, 1, d), jnp.float32),
            pltpu.VMEM((1, _BW, d), jnp.float32),
            pltpu.VMEM((1, _BW, d), jnp.float32),
            pltpu.VMEM((_CHUNK, n_gen, d), jnp.float32),
            pltpu.VMEM((n_gen, _CHUNK, d), jnp.float32),
            pltpu.VMEM_SHARED((n_sh, _TW, d), jnp.float32),
            pltpu.SemaphoreType.DMA,
            pltpu.SemaphoreType.DMA,
            pltpu.SemaphoreType.DMA,
            pltpu.SemaphoreType.DMA,
        ],
    )
    def k(table_hbm, label_hbm, tmpl5_hbm, crows_hbm, out_hbm,
          idx_v, crows_v, bc0_v, bc1_v, stage_v, genbuf_v, shared,
          gsem, wsem, tsem, vsem):
        cid = lax.axis_index("c")
        sid = lax.axis_index("s")
        wid = sid * 2 + cid
        base = pl.multiple_of(wid * b_per_w, b_per_w)
        pltpu.sync_copy(label_hbm.at[pl.ds(base, b_per_w)], idx_v)
        pltpu.sync_copy(crows_hbm.at[pl.ds(wid * 2, 2)], crows_v)

        @pl.when(sid < n_sh)
        def _load_shared():
            pltpu.sync_copy(tmpl5_hbm.at[pl.ds(sid, 1)],
                            shared.at[pl.ds(sid, 1)])

        for i, bc in enumerate((bc0_v, bc1_v)):
            vals = [crows_v[i, 0, pl.ds(l * _LANES, _LANES)]
                    for l in range(lanes_per_row)]
            for r in range(_BW):
                for l in range(lanes_per_row):
                    bc[0, r, pl.ds(l * _LANES, _LANES)] = vals[l]

        own_waits = []
        for i, bc in enumerate((bc0_v, bc1_v)):
            t_own = own_t0 + wid * 2 + i
            for h in range(b // _BW):
                dsc = pltpu.make_async_copy(
                    bc, out_hbm.at[pl.ds(t_own, 1), pl.ds(h * _BW, _BW), :],
                    wsem)
                dsc.start()
                own_waits.append(dsc)

        plsc.subcore_barrier()

        for h in range(b_per_w // _TW):
            dst_b = pl.ds(base + h * _TW, _TW)
            pltpu.make_async_copy(
                shared.at[pl.ds(0, 1)],
                out_hbm.at[pl.ds(0, 1), dst_b, :], tsem).start()
            pltpu.make_async_copy(
                shared.at[pl.ds(1, n_sh - 1)],
                out_hbm.at[pl.ds(1 + n_gen, n_sh - 1), dst_b, :],
                tsem).start()

        def chunk_body(c, carry):
            coff = pl.multiple_of(c * _CHUNK, _CHUNK)
            pltpu.async_copy(
                table_hbm.at[idx_v.at[pl.ds(coff, _CHUNK)]],
                stage_v, gsem).wait()

            @pl.when(c > 0)
            def _drain_prev():
                pltpu.make_async_copy(
                    table_hbm.at[pl.ds(0, _CHUNK)], genbuf_v, vsem).wait()

            for r in range(_CHUNK):
                for j in range(n_gen):
                    for l in range(lanes_per_row):
                        genbuf_v[j, r, pl.ds(l * _LANES, _LANES)] = (
                            stage_v[r, j, pl.ds(l * _LANES, _LANES)])
            pltpu.make_async_copy(
                genbuf_v,
                out_hbm.at[pl.ds(1, n_gen), pl.ds(base + coff, _CHUNK), :],
                vsem).start()
            return carry

        lax.fori_loop(0, n_chunks, chunk_body, 0)

        pltpu.make_async_copy(
            table_hbm.at[pl.ds(0, _CHUNK)], genbuf_v, vsem).wait()
        for dsc in own_waits:
            dsc.wait()
        for _ in range(b_per_w // _TW):
            pltpu.make_async_copy(tmpl5_hbm, shared, tsem).wait()

    return k(table, labels, tmpl5, crows)


def kernel(label, ctx_generic, ctx_modality, ctx_platform,
           token_prefix, token_suffix):
    n_gen = ctx_generic.shape[1]
    d = ctx_generic.shape[2]
    n_zero = ctx_modality.shape[1] + ctx_platform.shape[1]
    n_suf = token_suffix.shape[1]
    n_own = 2 * _NUM_WORKERS
    n_zero_own = n_own - n_suf
    n_zero_sh = n_zero - n_zero_own

    tmpl5 = jnp.concatenate([
        jnp.broadcast_to(token_prefix.astype(jnp.float32),
                         (token_prefix.shape[1], _TW, d)),
        jnp.zeros((n_zero_sh, _TW, d), jnp.float32),
    ], axis=0)
    crows = jnp.concatenate([
        jnp.zeros((n_zero_own, 1, d), jnp.float32),
        jnp.transpose(token_suffix.astype(jnp.float32), (1, 0, 2)),
    ], axis=0)
    slabbed = _sc_prompt_fill(ctx_generic, label.astype(jnp.int32),
                              tmpl5, crows)
    return jnp.transpose(slabbed, (1, 0, 2))
